# baseline (speedup 1.0000x reference)
import functools

import jax
import jax.numpy as jnp
from jax import lax
from jax.experimental import pallas as pl
from jax.experimental.pallas import tpu as pltpu
from jax.experimental.pallas import tpu_sc as plsc

N = 10000
E = 320000
F = 128
R = 32
L = 3
CUTOFF = 5.0
MAXZ = 101

NC = 2
NS = 16
NPASS = 2
PROWS = 5120
PCHUNK = PROWS // NS
P0 = 5000
DISCARD = PROWS - 1
EPAD = 327680
EPC = EPAD // NC
EPS = EPC // NS
BSC = 128

GW = 4
GBS = 80

BN = 1000
BE = 4096

_PREC = lax.Precision.HIGHEST
_WIDTH = CUTOFF / (R - 1)


@functools.cache
def _vector_mesh():
    return plsc.VectorSubcoreMesh(
        core_axis_name="core", subcore_axis_name="subcore"
    )


def _embed_body(z_ref, emb_ref, win_ref, x_ref, h_ref):
    z = z_ref[...]
    oh = (z == lax.broadcasted_iota(jnp.int32, (BN, 128), 1)).astype(jnp.float32)
    x = jnp.dot(oh, emb_ref[...], precision=_PREC,
                preferred_element_type=jnp.float32)
    x_ref[...] = x
    h_ref[...] = jnp.dot(x, win_ref[...], precision=_PREC,
                         preferred_element_type=jnp.float32)


def _embed_call(z2, emb_pad, win0):
    return pl.pallas_call(
        _embed_body,
        grid=(N // BN,),
        in_specs=[
            pl.BlockSpec((BN, 1), lambda i: (i, 0)),
            pl.BlockSpec((128, F), lambda i: (0, 0)),
            pl.BlockSpec((F, F), lambda i: (0, 0)),
        ],
        out_specs=[
            pl.BlockSpec((BN, F), lambda i: (i, 0)),
            pl.BlockSpec((BN, F), lambda i: (i, 0)),
        ],
        out_shape=[
            jax.ShapeDtypeStruct((N, F), jnp.float32),
            jax.ShapeDtypeStruct((N, F), jnp.float32),
        ],
    )(z2, emb_pad, win0)


def _rbf_body(d_ref, f_ref, fc_ref):
    d = d_ref[...]
    offs = lax.broadcasted_iota(jnp.int32, (1, R), 1).astype(jnp.float32) * _WIDTH
    f_ref[...] = jnp.exp((-0.5 / (_WIDTH * _WIDTH)) * (d - offs) ** 2)
    dc = jnp.minimum(d, CUTOFF)
    fc_ref[...] = 0.5 * (jnp.cos(jnp.pi * dc / CUTOFF) + 1.0)


def _rbf_call(d_ij):
    return pl.pallas_call(
        _rbf_body,
        grid=(EPAD // BE,),
        in_specs=[pl.BlockSpec((BE, 1), lambda e: (e, 0))],
        out_specs=[
            pl.BlockSpec((BE, R), lambda e: (e, 0)),
            pl.BlockSpec((BE, 1), lambda e: (e, 0)),
        ],
        out_shape=[
            jax.ShapeDtypeStruct((EPAD, R), jnp.float32),
            jax.ShapeDtypeStruct((EPAD, 1), jnp.float32),
        ],
    )(d_ij)


def _wij_body(f_ref, fc_ref, wf1_ref, bf1_ref, wf2_ref, bf2_ref, wij_ref):
    hmid = jnp.maximum(
        jnp.dot(f_ref[...], wf1_ref[0],
                preferred_element_type=jnp.float32) + bf1_ref[0], 0.0)
    wij = jnp.dot(hmid, wf2_ref[0],
                  preferred_element_type=jnp.float32) + bf2_ref[0]
    wij_ref[0] = wij * fc_ref[...]


def _wij_call(f, fc, Wf1, bf1, Wf2, bf2):
    return pl.pallas_call(
        _wij_body,
        grid=(L, EPAD // BE),
        in_specs=[
            pl.BlockSpec((BE, R), lambda l, e: (e, 0)),
            pl.BlockSpec((BE, 1), lambda l, e: (e, 0)),
            pl.BlockSpec((1, R, F), lambda l, e: (l, 0, 0)),
            pl.BlockSpec((1, 1, F), lambda l, e: (l, 0, 0)),
            pl.BlockSpec((1, F, F), lambda l, e: (l, 0, 0)),
            pl.BlockSpec((1, 1, F), lambda l, e: (l, 0, 0)),
        ],
        out_specs=pl.BlockSpec((1, BE, F), lambda l, e: (l, e, 0)),
        out_shape=jax.ShapeDtypeStruct((L, EPAD, F), jnp.float32),
    )(f, fc, Wf1, bf1.reshape(L, 1, F), Wf2, bf2.reshape(L, 1, F))


def _mul_body(xj_ref, wij_ref, xij_ref):
    xij_ref[...] = xj_ref[...] * wij_ref[0]


def _mul_call(xj, wij_all, l):
    return pl.pallas_call(
        _mul_body,
        grid=(EPAD // BE,),
        in_specs=[
            pl.BlockSpec((BE, F), lambda e: (e, 0)),
            pl.BlockSpec((1, BE, F), lambda e: (l, e, 0)),
        ],
        out_specs=pl.BlockSpec((BE, F), lambda e: (e, 0)),
        out_shape=jax.ShapeDtypeStruct((EPAD, F), jnp.float32),
    )(xj, wij_all)


def _update_body(agg_ref, x_ref, wo1_ref, bo1_ref, wo2_ref, bo2_ref,
                 win_ref, x_out, h_out):
    a = agg_ref[0, 0] + agg_ref[1, 0]
    t = jnp.maximum(
        jnp.dot(a, wo1_ref[...], precision=_PREC,
                preferred_element_type=jnp.float32) + bo1_ref[...], 0.0)
    v = jnp.dot(t, wo2_ref[...], precision=_PREC,
                preferred_element_type=jnp.float32) + bo2_ref[...]
    xn = x_ref[...] + v
    x_out[...] = xn
    h_out[...] = jnp.dot(xn, win_ref[...], precision=_PREC,
                         preferred_element_type=jnp.float32)


def _update_final_body(agg_ref, x_ref, wo1_ref, bo1_ref, wo2_ref, bo2_ref,
                       x_out):
    a = agg_ref[0, 0] + agg_ref[1, 0]
    t = jnp.maximum(
        jnp.dot(a, wo1_ref[...], precision=_PREC,
                preferred_element_type=jnp.float32) + bo1_ref[...], 0.0)
    v = jnp.dot(t, wo2_ref[...], precision=_PREC,
                preferred_element_type=jnp.float32) + bo2_ref[...]
    x_out[...] = x_ref[...] + v


_UPD_IN_SPECS = [
    pl.BlockSpec((2, 1, BN, F), lambda i: (0, i // 5, i % 5, 0)),
    pl.BlockSpec((BN, F), lambda i: (i, 0)),
    pl.BlockSpec((F, F), lambda i: (0, 0)),
    pl.BlockSpec((1, F), lambda i: (0, 0)),
    pl.BlockSpec((F, F), lambda i: (0, 0)),
    pl.BlockSpec((1, F), lambda i: (0, 0)),
]


def _update_call(agg, x, wo1, bo1, wo2, bo2, win_next):
    return pl.pallas_call(
        _update_body,
        grid=(N // BN,),
        in_specs=_UPD_IN_SPECS + [pl.BlockSpec((F, F), lambda i: (0, 0))],
        out_specs=[
            pl.BlockSpec((BN, F), lambda i: (i, 0)),
            pl.BlockSpec((BN, F), lambda i: (i, 0)),
        ],
        out_shape=[
            jax.ShapeDtypeStruct((N, F), jnp.float32),
            jax.ShapeDtypeStruct((N, F), jnp.float32),
        ],
    )(agg, x, wo1, bo1, wo2, bo2, win_next)


def _update_final_call(agg, x, wo1, bo1, wo2, bo2):
    return pl.pallas_call(
        _update_final_body,
        grid=(N // BN,),
        in_specs=_UPD_IN_SPECS,
        out_specs=pl.BlockSpec((BN, F), lambda i: (i, 0)),
        out_shape=jax.ShapeDtypeStruct((N, F), jnp.float32),
    )(agg, x, wo1, bo1, wo2, bo2)


def _sc_gather(h, idxj3):
    @functools.partial(
        pl.kernel,
        out_type=jax.ShapeDtypeStruct((EPAD, F), jnp.float32),
        mesh=_vector_mesh(),
        scratch_types=[
            pltpu.SemaphoreType.DMA,
            pltpu.SemaphoreType.DMA,
            pltpu.SemaphoreType.DMA,
            pltpu.SemaphoreType.DMA,
        ],
    )
    def gk(h_hbm, idx_hbm, o_hbm, sem0, sem1, sem2, sem3):
        sems = (sem0, sem1, sem2, sem3)

        def body(idx_vmem, o_vmem):
            ds = [pltpu.async_copy(h_hbm.at[idx_vmem.at[0, k]],
                                   o_vmem.at[pl.ds(k * GBS, GBS)], sems[k])
                  for k in range(GW)]
            for d in ds:
                d.wait()

        pltpu.emit_pipeline(
            body,
            grid=(EPAD // (GW * GBS),),
            in_specs=[pl.BlockSpec((1, GW, GBS), lambda i: (i, 0, 0))],
            out_specs=[pl.BlockSpec((GW * GBS, F), lambda i: (i, 0))],
            core_axis_name=("core", "subcore"),
            dimension_semantics=(pltpu.PARALLEL,),
        )(idx_hbm, o_hbm)

    return gk(h, idxj3)


def _sc_scatter(xij, idxp0, idxp1, zrows):
    @functools.partial(
        pl.kernel,
        out_type=jax.ShapeDtypeStruct((NC, NPASS, PROWS, F), jnp.float32),
        mesh=_vector_mesh(),
        scratch_types=[
            pltpu.VMEM_SHARED((PROWS, F), jnp.float32),
            pltpu.SemaphoreType.DMA,
            pltpu.SemaphoreType.DMA,
        ],
    )
    def sk(x_hbm, i0_hbm, i1_hbm, z_hbm, o_hbm, agg_sh, sem0, sem1):
        c = lax.axis_index("core")
        s = lax.axis_index("subcore")
        for p, idx_hbm in enumerate((i0_hbm, i1_hbm)):
            pltpu.sync_copy(z_hbm, agg_sh.at[pl.ds(s * PCHUNK, PCHUNK)])
            plsc.subcore_barrier()

            def body(x_vmem, idx_vmem):
                a0 = pltpu.async_copy(x_vmem.at[pl.ds(0, BSC)],
                                      agg_sh.at[idx_vmem.at[0, 0]],
                                      sem0, add=True)
                a1 = pltpu.async_copy(x_vmem.at[pl.ds(BSC, BSC)],
                                      agg_sh.at[idx_vmem.at[0, 1]],
                                      sem1, add=True)
                a0.wait()
                a1.wait()

            pltpu.emit_pipeline(
                body,
                grid=(EPAD // (2 * BSC),),
                in_specs=[pl.BlockSpec((2 * BSC, F), lambda i: (i, 0)),
                          pl.BlockSpec((1, 2, BSC), lambda i: (i, 0, 0))],
                out_specs=[],
                core_axis_name=("core", "subcore"),
                dimension_semantics=(pltpu.PARALLEL,),
            )(x_hbm, idx_hbm)

            plsc.subcore_barrier()
            pltpu.sync_copy(agg_sh.at[pl.ds(s * PCHUNK, PCHUNK)],
                            o_hbm.at[c, p, pl.ds(s * PCHUNK, PCHUNK)])
            plsc.subcore_barrier()

    return sk(xij, idxp0, idxp1, zrows)


def kernel(d_ij, embedding, W_in, Wf1, bf1, Wf2, bf2, Wo1, bo1, Wo2, bo2,
           pair_indices, atomic_numbers):
    emb_pad = jnp.pad(embedding.astype(jnp.float32), ((0, 128 - MAXZ), (0, 0)))
    z2 = atomic_numbers.astype(jnp.int32).reshape(N, 1)
    pad_e = EPAD - E
    idx_i = jnp.pad(pair_indices[0].astype(jnp.int32), (0, pad_e),
                    constant_values=P0 + DISCARD)
    idx_j = jnp.pad(pair_indices[1].astype(jnp.int32), (0, pad_e))
    blk3 = (EPAD // (2 * BSC), 2, BSC)
    idxj3 = idx_j.reshape(EPAD // (GW * GBS), GW, GBS)
    idxp0 = jnp.where(idx_i < P0, idx_i, DISCARD).reshape(blk3)
    idxp1 = jnp.where(idx_i >= P0, idx_i - P0, DISCARD).reshape(blk3)
    zrows = jnp.zeros((PCHUNK, F), jnp.float32)
    d_ij = jnp.pad(d_ij.astype(jnp.float32), ((0, pad_e), (0, 0)))

    f, fc = _rbf_call(d_ij)
    wij_all = _wij_call(f, fc, Wf1, bf1, Wf2, bf2)
    x, h = _embed_call(z2, emb_pad, W_in[0])
    for l in range(L):
        xj = _sc_gather(h, idxj3)
        xij = _mul_call(xj, wij_all, l)
        aggp = _sc_scatter(xij, idxp0, idxp1, zrows)
        if l < L - 1:
            x, h = _update_call(aggp, x, Wo1[l], bo1[l][None], Wo2[l],
                                bo2[l][None], W_in[l + 1])
        else:
            x = _update_final_call(aggp, x, Wo1[l], bo1[l][None], Wo2[l],
                                   bo2[l][None])
    return x


# speedup vs baseline: 1.0358x; 1.0358x over previous
import functools

import jax
import jax.numpy as jnp
from jax import lax
from jax.experimental import pallas as pl
from jax.experimental.pallas import tpu as pltpu
from jax.experimental.pallas import tpu_sc as plsc

N = 10000
E = 320000
F = 128
R = 32
L = 3
CUTOFF = 5.0
MAXZ = 101

NC = 2
NS = 16
NPASS = 2
PROWS = 5120
PCHUNK = PROWS // NS
P0 = 5000
DISCARD = PROWS - 1
EPAD = 327680
EPC = EPAD // NC
EPS = EPC // NS
BSC = 128

GW = 2
GBS = 128

BN = 1000
BE = 4096

_PREC = lax.Precision.HIGHEST
_WIDTH = CUTOFF / (R - 1)


@functools.cache
def _vector_mesh():
    return plsc.VectorSubcoreMesh(
        core_axis_name="core", subcore_axis_name="subcore"
    )


def _embed_body(z_ref, emb_ref, win_ref, x_ref, h_ref):
    z = z_ref[...]
    oh = (z == lax.broadcasted_iota(jnp.int32, (BN, 128), 1)).astype(jnp.float32)
    x = jnp.dot(oh, emb_ref[...], precision=_PREC,
                preferred_element_type=jnp.float32)
    x_ref[...] = x
    h_ref[...] = jnp.dot(x, win_ref[...], precision=_PREC,
                         preferred_element_type=jnp.float32)


def _embed_call(z2, emb_pad, win0):
    return pl.pallas_call(
        _embed_body,
        grid=(N // BN,),
        in_specs=[
            pl.BlockSpec((BN, 1), lambda i: (i, 0)),
            pl.BlockSpec((128, F), lambda i: (0, 0)),
            pl.BlockSpec((F, F), lambda i: (0, 0)),
        ],
        out_specs=[
            pl.BlockSpec((BN, F), lambda i: (i, 0)),
            pl.BlockSpec((BN, F), lambda i: (i, 0)),
        ],
        out_shape=[
            jax.ShapeDtypeStruct((N, F), jnp.float32),
            jax.ShapeDtypeStruct((N, F), jnp.float32),
        ],
    )(z2, emb_pad, win0)


def _rbf_body(d_ref, f_ref, fc_ref):
    d = d_ref[...]
    offs = lax.broadcasted_iota(jnp.int32, (1, R), 1).astype(jnp.float32) * _WIDTH
    f_ref[...] = jnp.exp((-0.5 / (_WIDTH * _WIDTH)) * (d - offs) ** 2)
    dc = jnp.minimum(d, CUTOFF)
    fc_ref[...] = 0.5 * (jnp.cos(jnp.pi * dc / CUTOFF) + 1.0)


def _rbf_call(d_ij):
    return pl.pallas_call(
        _rbf_body,
        grid=(EPAD // BE,),
        in_specs=[pl.BlockSpec((BE, 1), lambda e: (e, 0))],
        out_specs=[
            pl.BlockSpec((BE, R), lambda e: (e, 0)),
            pl.BlockSpec((BE, 1), lambda e: (e, 0)),
        ],
        out_shape=[
            jax.ShapeDtypeStruct((EPAD, R), jnp.float32),
            jax.ShapeDtypeStruct((EPAD, 1), jnp.float32),
        ],
    )(d_ij)


def _wij_body(f_ref, fc_ref, wf1_ref, bf1_ref, wf2_ref, bf2_ref, wij_ref):
    hmid = jnp.maximum(
        jnp.dot(f_ref[...], wf1_ref[0],
                preferred_element_type=jnp.float32) + bf1_ref[0], 0.0)
    wij = jnp.dot(hmid, wf2_ref[0],
                  preferred_element_type=jnp.float32) + bf2_ref[0]
    wij_ref[0] = wij * fc_ref[...]


def _wij_call(f, fc, Wf1, bf1, Wf2, bf2):
    return pl.pallas_call(
        _wij_body,
        grid=(L, EPAD // BE),
        in_specs=[
            pl.BlockSpec((BE, R), lambda l, e: (e, 0)),
            pl.BlockSpec((BE, 1), lambda l, e: (e, 0)),
            pl.BlockSpec((1, R, F), lambda l, e: (l, 0, 0)),
            pl.BlockSpec((1, 1, F), lambda l, e: (l, 0, 0)),
            pl.BlockSpec((1, F, F), lambda l, e: (l, 0, 0)),
            pl.BlockSpec((1, 1, F), lambda l, e: (l, 0, 0)),
        ],
        out_specs=pl.BlockSpec((1, BE, F), lambda l, e: (l, e, 0)),
        out_shape=jax.ShapeDtypeStruct((L, EPAD, F), jnp.float32),
    )(f, fc, Wf1, bf1.reshape(L, 1, F), Wf2, bf2.reshape(L, 1, F))


def _mul_body(xj_ref, wij_ref, xij_ref):
    xij_ref[...] = xj_ref[...] * wij_ref[0]


def _mul_call(xj, wij_all, l):
    return pl.pallas_call(
        _mul_body,
        grid=(EPAD // BE,),
        in_specs=[
            pl.BlockSpec((BE, F), lambda e: (e, 0)),
            pl.BlockSpec((1, BE, F), lambda e: (l, e, 0)),
        ],
        out_specs=pl.BlockSpec((BE, F), lambda e: (e, 0)),
        out_shape=jax.ShapeDtypeStruct((EPAD, F), jnp.float32),
    )(xj, wij_all)


def _update_body(agg_ref, x_ref, wo1_ref, bo1_ref, wo2_ref, bo2_ref,
                 win_ref, x_out, h_out):
    a = agg_ref[0, 0] + agg_ref[1, 0]
    t = jnp.maximum(
        jnp.dot(a, wo1_ref[...], precision=_PREC,
                preferred_element_type=jnp.float32) + bo1_ref[...], 0.0)
    v = jnp.dot(t, wo2_ref[...], precision=_PREC,
                preferred_element_type=jnp.float32) + bo2_ref[...]
    xn = x_ref[...] + v
    x_out[...] = xn
    h_out[...] = jnp.dot(xn, win_ref[...], precision=_PREC,
                         preferred_element_type=jnp.float32)


def _update_final_body(agg_ref, x_ref, wo1_ref, bo1_ref, wo2_ref, bo2_ref,
                       x_out):
    a = agg_ref[0, 0] + agg_ref[1, 0]
    t = jnp.maximum(
        jnp.dot(a, wo1_ref[...], precision=_PREC,
                preferred_element_type=jnp.float32) + bo1_ref[...], 0.0)
    v = jnp.dot(t, wo2_ref[...], precision=_PREC,
                preferred_element_type=jnp.float32) + bo2_ref[...]
    x_out[...] = x_ref[...] + v


_UPD_IN_SPECS = [
    pl.BlockSpec((2, 1, BN, F), lambda i: (0, i // 5, i % 5, 0)),
    pl.BlockSpec((BN, F), lambda i: (i, 0)),
    pl.BlockSpec((F, F), lambda i: (0, 0)),
    pl.BlockSpec((1, F), lambda i: (0, 0)),
    pl.BlockSpec((F, F), lambda i: (0, 0)),
    pl.BlockSpec((1, F), lambda i: (0, 0)),
]


def _update_call(agg, x, wo1, bo1, wo2, bo2, win_next):
    return pl.pallas_call(
        _update_body,
        grid=(N // BN,),
        in_specs=_UPD_IN_SPECS + [pl.BlockSpec((F, F), lambda i: (0, 0))],
        out_specs=[
            pl.BlockSpec((BN, F), lambda i: (i, 0)),
            pl.BlockSpec((BN, F), lambda i: (i, 0)),
        ],
        out_shape=[
            jax.ShapeDtypeStruct((N, F), jnp.float32),
            jax.ShapeDtypeStruct((N, F), jnp.float32),
        ],
    )(agg, x, wo1, bo1, wo2, bo2, win_next)


def _update_final_call(agg, x, wo1, bo1, wo2, bo2):
    return pl.pallas_call(
        _update_final_body,
        grid=(N // BN,),
        in_specs=_UPD_IN_SPECS,
        out_specs=pl.BlockSpec((BN, F), lambda i: (i, 0)),
        out_shape=jax.ShapeDtypeStruct((N, F), jnp.float32),
    )(agg, x, wo1, bo1, wo2, bo2)


def _sc_gather(h, idxj3):
    @functools.partial(
        pl.kernel,
        out_type=jax.ShapeDtypeStruct((EPAD, F), jnp.float32),
        mesh=_vector_mesh(),
        scratch_types=[
            pltpu.SemaphoreType.DMA,
            pltpu.SemaphoreType.DMA,
        ],
    )
    def gk(h_hbm, idx_hbm, o_hbm, sem0, sem1):
        sems = (sem0, sem1)

        def body(idx_vmem, o_vmem):
            ds = [pltpu.async_copy(h_hbm.at[idx_vmem.at[0, k]],
                                   o_vmem.at[pl.ds(k * GBS, GBS)], sems[k])
                  for k in range(GW)]
            for d in ds:
                d.wait()

        pltpu.emit_pipeline(
            body,
            grid=(EPAD // (GW * GBS),),
            in_specs=[pl.BlockSpec((1, GW, GBS), lambda i: (i, 0, 0))],
            out_specs=[pl.BlockSpec((GW * GBS, F), lambda i: (i, 0))],
            core_axis_name=("core", "subcore"),
            dimension_semantics=(pltpu.PARALLEL,),
        )(idx_hbm, o_hbm)

    return gk(h, idxj3)


def _sc_scatter(xij, idxp0, idxp1, zrows):
    @functools.partial(
        pl.kernel,
        out_type=jax.ShapeDtypeStruct((NC, NPASS, PROWS, F), jnp.float32),
        mesh=_vector_mesh(),
        scratch_types=[
            pltpu.VMEM_SHARED((PROWS, F), jnp.float32),
            pltpu.SemaphoreType.DMA,
            pltpu.SemaphoreType.DMA,
        ],
    )
    def sk(x_hbm, i0_hbm, i1_hbm, z_hbm, o_hbm, agg_sh, sem0, sem1):
        c = lax.axis_index("core")
        s = lax.axis_index("subcore")
        for p, idx_hbm in enumerate((i0_hbm, i1_hbm)):
            pltpu.sync_copy(z_hbm, agg_sh.at[pl.ds(s * PCHUNK, PCHUNK)])
            plsc.subcore_barrier()

            def body(x_vmem, idx_vmem):
                a0 = pltpu.async_copy(x_vmem.at[pl.ds(0, BSC)],
                                      agg_sh.at[idx_vmem.at[0, 0]],
                                      sem0, add=True)
                a1 = pltpu.async_copy(x_vmem.at[pl.ds(BSC, BSC)],
                                      agg_sh.at[idx_vmem.at[0, 1]],
                                      sem1, add=True)
                a0.wait()
                a1.wait()

            pltpu.emit_pipeline(
                body,
                grid=(EPAD // (2 * BSC),),
                in_specs=[pl.BlockSpec((2 * BSC, F), lambda i: (i, 0)),
                          pl.BlockSpec((1, 2, BSC), lambda i: (i, 0, 0))],
                out_specs=[],
                core_axis_name=("core", "subcore"),
                dimension_semantics=(pltpu.PARALLEL,),
            )(x_hbm, idx_hbm)

            plsc.subcore_barrier()
            pltpu.sync_copy(agg_sh.at[pl.ds(s * PCHUNK, PCHUNK)],
                            o_hbm.at[c, p, pl.ds(s * PCHUNK, PCHUNK)])
            plsc.subcore_barrier()

    return sk(xij, idxp0, idxp1, zrows)


def kernel(d_ij, embedding, W_in, Wf1, bf1, Wf2, bf2, Wo1, bo1, Wo2, bo2,
           pair_indices, atomic_numbers):
    emb_pad = jnp.pad(embedding.astype(jnp.float32), ((0, 128 - MAXZ), (0, 0)))
    z2 = atomic_numbers.astype(jnp.int32).reshape(N, 1)
    pad_e = EPAD - E
    idx_i = jnp.pad(pair_indices[0].astype(jnp.int32), (0, pad_e),
                    constant_values=P0 + DISCARD)
    idx_j = jnp.pad(pair_indices[1].astype(jnp.int32), (0, pad_e))
    blk3 = (EPAD // (2 * BSC), 2, BSC)
    idxj3 = idx_j.reshape(EPAD // (GW * GBS), GW, GBS)
    idxp0 = jnp.where(idx_i < P0, idx_i, DISCARD).reshape(blk3)
    idxp1 = jnp.where(idx_i >= P0, idx_i - P0, DISCARD).reshape(blk3)
    zrows = jnp.zeros((PCHUNK, F), jnp.float32)
    d_ij = jnp.pad(d_ij.astype(jnp.float32), ((0, pad_e), (0, 0)))

    f, fc = _rbf_call(d_ij)
    wij_all = _wij_call(f, fc, Wf1, bf1, Wf2, bf2)
    x, h = _embed_call(z2, emb_pad, W_in[0])
    for l in range(L):
        xj = _sc_gather(h, idxj3)
        xij = _mul_call(xj, wij_all, l)
        aggp = _sc_scatter(xij, idxp0, idxp1, zrows)
        if l < L - 1:
            x, h = _update_call(aggp, x, Wo1[l], bo1[l][None], Wo2[l],
                                bo2[l][None], W_in[l + 1])
        else:
            x = _update_final_call(aggp, x, Wo1[l], bo1[l][None], Wo2[l],
                                   bo2[l][None])
    return x
